# Initial kernel scaffold; baseline (speedup 1.0000x reference)
import jax
import jax.numpy as jnp
from jax.experimental import pallas as pl


def kernel(in_feat, edge_index, W1, b1, W2, b2):
    raise NotImplementedError("write your pallas kernel here")


# speedup vs baseline: 5.2268x; 5.2268x over previous
import functools

import jax
import jax.numpy as jnp
from jax import lax
from jax.experimental import pallas as pl
from jax.experimental.pallas import tpu as pltpu
from jax.experimental.pallas import tpu_sc as plsc

N = 10000
E = 160000
D_IN = 256
D_H = 256
D_OUT = 64

NC = 2
NS = 16
NP = 10240
ROWS_PER_TILE = NP // NS
CHUNK = 128

_MESH = plsc.VectorSubcoreMesh(core_axis_name="c", subcore_axis_name="s")

_ZERO_SLICES = ((0, 128), (128, 128), (256, 128), (384, 128), (512, 128))


def _fill_rows(ref, nrows, ncols, value):
    vec = jnp.full((16,), value, jnp.float32)

    def body(i, _):
        for j in range(ncols // 16):
            ref[i, pl.ds(j * 16, 16)] = vec
        return 0

    lax.fori_loop(0, nrows, body, 0)


def _deg_body(ei_ref, out_ref, idx_v, hist, tmp, sums, vbuf, shared):
    c = lax.axis_index("c")
    s = lax.axis_index("s")
    zero16 = jnp.zeros((16,), jnp.float32)
    one16 = jnp.ones((16,), jnp.float32)

    def zh(i, _):
        hist[pl.ds(i * 16, 16)] = zero16
        return 0

    lax.fori_loop(0, NP // 16, zh, 0)

    epb = E // NS
    ebase = c * E + s * epb

    def step(k, _):
        off = pl.multiple_of(ebase + k * CHUNK, 8)
        pltpu.sync_copy(ei_ref.at[pl.ds(off, CHUNK)], idx_v)
        for j in range(CHUNK // 16):
            v = idx_v[pl.ds(j * 16, 16)]
            plsc.addupdate_scatter(hist, [v], one16)
        return 0

    nfull = epb // CHUNK
    lax.fori_loop(0, nfull, step, 0)
    rem = epb - nfull * CHUNK
    off = pl.multiple_of(ebase + nfull * CHUNK, 8)
    pltpu.sync_copy(ei_ref.at[pl.ds(off, rem)], idx_v.at[pl.ds(0, rem)])
    plsc.addupdate_scatter(hist, [idx_v[pl.ds(0, rem)]], one16)

    pltpu.sync_copy(hist, shared.at[s])
    plsc.subcore_barrier()
    base = s * ROWS_PER_TILE

    def zsum(g, _):
        sums[pl.ds(g * 16, 16)] = zero16
        return 0

    lax.fori_loop(0, ROWS_PER_TILE // 16, zsum, 0)
    for t in range(NS):
        pltpu.sync_copy(shared.at[t, pl.ds(base, ROWS_PER_TILE)], tmp)

        def addg(g, _):
            sums[pl.ds(g * 16, 16)] = sums[pl.ds(g * 16, 16)] + tmp[pl.ds(g * 16, 16)]
            return 0

        lax.fori_loop(0, ROWS_PER_TILE // 16, addg, 0)

    iota16 = lax.iota(jnp.int32, 16)

    def wg(g, _):
        v = sums[pl.ds(g * 16, 16)]
        plsc.store_scatter(vbuf, [iota16 * 16 + g * 256], v)
        return 0

    lax.fori_loop(0, ROWS_PER_TILE // 16, wg, 0)
    obase = c * (NP * 16) + s * (ROWS_PER_TILE * 16)
    pltpu.sync_copy(vbuf, out_ref.at[pl.ds(obase, ROWS_PER_TILE * 16)])


_sc_degree = functools.partial(
    pl.kernel,
    _deg_body,
    out_type=jax.ShapeDtypeStruct((2 * NP * 16,), jnp.float32),
    mesh=_MESH,
    scratch_types=[
        pltpu.VMEM((CHUNK,), jnp.int32),
        pltpu.VMEM((NP,), jnp.float32),
        pltpu.VMEM((ROWS_PER_TILE,), jnp.float32),
        pltpu.VMEM((ROWS_PER_TILE,), jnp.float32),
        pltpu.VMEM((ROWS_PER_TILE * 16,), jnp.float32),
        pltpu.VMEM_SHARED((NS, NP), jnp.float32),
    ],
    compiler_params=pltpu.CompilerParams(needs_layout_passes=False),
)


def _spmm1_body(x2_ref, ei_ref, out_ref, rows_v, sidx, didx, sidx_r, didx_r,
                acc, sem):
    c = lax.axis_index("c")
    s = lax.axis_index("s")
    _fill_rows(rows_v, CHUNK, 128, 0.0)
    base_row = s * ROWS_PER_TILE
    for r0, nr in _ZERO_SLICES:
        pltpu.sync_copy(rows_v.at[pl.ds(0, nr)], acc.at[pl.ds(base_row + r0, nr)])
    plsc.subcore_barrier()

    epb = E // NS
    ebase = s * epb
    cN = c * NP

    def step(k, _):
        off = pl.multiple_of(ebase + k * CHUNK, 8)
        pltpu.sync_copy(ei_ref.at[pl.ds(off, CHUNK)], sidx)
        pltpu.sync_copy(ei_ref.at[pl.ds(off + E, CHUNK)], didx)
        for j in range(CHUNK // 16):
            v = sidx[pl.ds(j * 16, 16)]
            sidx[pl.ds(j * 16, 16)] = v + cN
        pltpu.async_copy(x2_ref.at[sidx], rows_v, sem).wait()
        pltpu.sync_copy(rows_v, acc.at[didx], add=True)
        return 0

    nfull = epb // CHUNK
    lax.fori_loop(0, nfull, step, 0)
    rem = epb - nfull * CHUNK
    off = pl.multiple_of(ebase + nfull * CHUNK, 8)
    pltpu.sync_copy(ei_ref.at[pl.ds(off, rem)], sidx_r)
    pltpu.sync_copy(ei_ref.at[pl.ds(off + E, rem)], didx_r)
    v = sidx_r[pl.ds(0, 16)]
    sidx_r[pl.ds(0, 16)] = v + cN
    pltpu.async_copy(x2_ref.at[sidx_r], rows_v.at[pl.ds(0, rem)], sem).wait()
    pltpu.sync_copy(rows_v.at[pl.ds(0, rem)], acc.at[didx_r], add=True)

    plsc.subcore_barrier()
    pltpu.sync_copy(acc.at[pl.ds(base_row, ROWS_PER_TILE)],
                    out_ref.at[pl.ds(cN + base_row, ROWS_PER_TILE)])


_sc_spmm1 = functools.partial(
    pl.kernel,
    _spmm1_body,
    out_type=jax.ShapeDtypeStruct((2 * NP, 128), jnp.float32),
    mesh=_MESH,
    scratch_types=[
        pltpu.VMEM((CHUNK, 128), jnp.float32),
        pltpu.VMEM((CHUNK,), jnp.int32),
        pltpu.VMEM((CHUNK,), jnp.int32),
        pltpu.VMEM((16,), jnp.int32),
        pltpu.VMEM((16,), jnp.int32),
        pltpu.VMEM_SHARED((NP, 128), jnp.float32),
        pltpu.SemaphoreType.DMA,
    ],
)


def _spmm2_body(z_ref, ei_ref, out_ref, rows_v, sidx, didx, sidx_r, didx_r,
                acc, sem):
    c = lax.axis_index("c")
    s = lax.axis_index("s")
    _fill_rows(rows_v, CHUNK, 128, 0.0)
    base_row = s * ROWS_PER_TILE
    for r0, nr in _ZERO_SLICES:
        pltpu.sync_copy(rows_v.at[pl.ds(0, nr)], acc.at[pl.ds(base_row + r0, nr)])
    plsc.subcore_barrier()

    epb = E // (NC * NS)
    ebase = c * (E // NC) + s * epb

    def step(k, _):
        off = pl.multiple_of(ebase + k * CHUNK, 8)
        pltpu.sync_copy(ei_ref.at[pl.ds(off, CHUNK)], sidx)
        pltpu.sync_copy(ei_ref.at[pl.ds(off + E, CHUNK)], didx)
        pltpu.async_copy(z_ref.at[sidx], rows_v, sem).wait()
        pltpu.sync_copy(rows_v, acc.at[didx], add=True)
        return 0

    nfull = epb // CHUNK
    lax.fori_loop(0, nfull, step, 0)
    rem = epb - nfull * CHUNK
    off = pl.multiple_of(ebase + nfull * CHUNK, 8)
    pltpu.sync_copy(ei_ref.at[pl.ds(off, rem)], sidx_r)
    pltpu.sync_copy(ei_ref.at[pl.ds(off + E, rem)], didx_r)
    pltpu.async_copy(z_ref.at[sidx_r], rows_v.at[pl.ds(0, rem)], sem).wait()
    pltpu.sync_copy(rows_v.at[pl.ds(0, rem)], acc.at[didx_r], add=True)

    plsc.subcore_barrier()
    pltpu.sync_copy(acc.at[pl.ds(base_row, ROWS_PER_TILE)],
                    out_ref.at[pl.ds(c * NP + base_row, ROWS_PER_TILE)])


_sc_spmm2 = functools.partial(
    pl.kernel,
    _spmm2_body,
    out_type=jax.ShapeDtypeStruct((2 * NP, 128), jnp.float32),
    mesh=_MESH,
    scratch_types=[
        pltpu.VMEM((CHUNK, 128), jnp.float32),
        pltpu.VMEM((CHUNK,), jnp.int32),
        pltpu.VMEM((CHUNK,), jnp.int32),
        pltpu.VMEM((8,), jnp.int32),
        pltpu.VMEM((8,), jnp.int32),
        pltpu.VMEM_SHARED((NP, 128), jnp.float32),
        pltpu.SemaphoreType.DMA,
    ],
)


BN = 400


def _norms(deg_blk):
    ns = lax.rsqrt(jnp.maximum(deg_blk[0, :, 0:1], 1.0))
    nd = lax.rsqrt(jnp.maximum(deg_blk[1, :, 0:1], 1.0))
    return ns, nd


def _tc_prep_body(deg_ref, x_ref, o_ref):
    ns, _ = _norms(deg_ref[...])
    xn = x_ref[...] * ns
    o_ref[0] = xn[:, :128]
    o_ref[1] = xn[:, 128:]


def _tc_prep(deg2, x):
    return pl.pallas_call(
        _tc_prep_body,
        grid=(N // BN,),
        in_specs=[
            pl.BlockSpec((2, BN, 16), lambda i: (0, i, 0)),
            pl.BlockSpec((BN, D_IN), lambda i: (i, 0)),
        ],
        out_specs=pl.BlockSpec((2, BN, 128), lambda i: (0, i, 0)),
        out_shape=jax.ShapeDtypeStruct((2, NP, 128), jnp.float32),
    )(deg2, x)


def _tc_mid_body(deg_ref, a_ref, w1_ref, b1_ref, w2_ref, z_ref):
    ns, nd = _norms(deg_ref[...])
    a0 = a_ref[0] * nd
    a1 = a_ref[1] * nd
    h = jnp.dot(a0, w1_ref[0:128, :], preferred_element_type=jnp.float32)
    h += jnp.dot(a1, w1_ref[128:256, :], preferred_element_type=jnp.float32)
    h = jnp.maximum(h + b1_ref[...], 0.0)
    z_ref[...] = jnp.dot(h * ns, w2_ref[...], preferred_element_type=jnp.float32)


def _tc_mid(deg2, agg1, W1, b1, W2):
    return pl.pallas_call(
        _tc_mid_body,
        grid=(N // BN,),
        in_specs=[
            pl.BlockSpec((2, BN, 16), lambda i: (0, i, 0)),
            pl.BlockSpec((2, BN, 128), lambda i: (0, i, 0)),
            pl.BlockSpec((D_IN, D_H), lambda i: (0, 0)),
            pl.BlockSpec((1, D_H), lambda i: (0, 0)),
            pl.BlockSpec((D_H, 128), lambda i: (0, 0)),
        ],
        out_specs=pl.BlockSpec((BN, 128), lambda i: (i, 0)),
        out_shape=jax.ShapeDtypeStruct((NP, 128), jnp.float32),
    )(deg2, agg1, W1, b1, W2)


def _tc_fin_body(deg_ref, az_ref, b2_ref, h_ref, p_ref):
    _, nd = _norms(deg_ref[...])
    h = (az_ref[0] + az_ref[1])[:, :D_OUT] * nd + b2_ref[...]
    h_ref[...] = h

    @pl.when(pl.program_id(0) == 0)
    def _():
        p_ref[...] = jnp.zeros_like(p_ref)

    p_ref[...] += jnp.sum(h, axis=0, keepdims=True) * (1.0 / N)


def _tc_fin(deg2, aggz, b2):
    return pl.pallas_call(
        _tc_fin_body,
        grid=(N // BN,),
        in_specs=[
            pl.BlockSpec((2, BN, 16), lambda i: (0, i, 0)),
            pl.BlockSpec((2, BN, 128), lambda i: (0, i, 0)),
            pl.BlockSpec((1, D_OUT), lambda i: (0, 0)),
        ],
        out_specs=[
            pl.BlockSpec((BN, D_OUT), lambda i: (i, 0)),
            pl.BlockSpec((1, D_OUT), lambda i: (0, 0)),
        ],
        out_shape=[
            jax.ShapeDtypeStruct((N, D_OUT), jnp.float32),
            jax.ShapeDtypeStruct((1, D_OUT), jnp.float32),
        ],
    )(deg2, aggz, b2)


def kernel(in_feat, edge_index, W1, b1, W2, b2):
    ei = edge_index.astype(jnp.int32).reshape(2 * E)
    deg2 = _sc_degree()(ei).reshape(2, NP, 16)
    xs = _tc_prep(deg2, in_feat)
    agg1 = _sc_spmm1()(xs.reshape(2 * NP, 128), ei)
    W2p = jnp.pad(W2, ((0, 0), (0, 128 - D_OUT)))
    z = _tc_mid(deg2, agg1.reshape(2, NP, 128), W1, b1.reshape(1, D_H), W2p)
    aggz = _sc_spmm2()(z, ei)
    h2, pooled = _tc_fin(deg2, aggz.reshape(2, NP, 128), b2.reshape(1, D_OUT))
    return (h2, pooled)


# speedup vs baseline: 7.5717x; 1.4486x over previous
import functools

import jax
import jax.numpy as jnp
from jax import lax
from jax.experimental import pallas as pl
from jax.experimental.pallas import tpu as pltpu
from jax.experimental.pallas import tpu_sc as plsc

N = 10000
E = 160000
D_IN = 256
D_H = 256
D_OUT = 64

NC = 2
NS = 16
NP = 10240
ROWS_PER_TILE = NP // NS
CHUNK = 128

ER = E // CHUNK
ERP = 1280

_MESH = plsc.VectorSubcoreMesh(core_axis_name="c", subcore_axis_name="s")

_ZERO_SLICES = ((0, 128), (128, 128), (256, 128), (384, 128), (512, 128))


def _fill_rows(ref, nrows, ncols, value):
    vec = jnp.full((16,), value, jnp.float32)

    def body(i, _):
        for j in range(ncols // 16):
            ref[i, pl.ds(j * 16, 16)] = vec
        return 0

    lax.fori_loop(0, nrows, body, 0)


_BIG = 1024


def _deg_body(ei_ref, out_ref, iva, ivb, hist, tmp, sums, vbuf, shared,
              sia, sib):
    c = lax.axis_index("c")
    s = lax.axis_index("s")
    zero16 = jnp.zeros((16,), jnp.float32)
    one16 = jnp.ones((16,), jnp.float32)

    def zh(i, _):
        hist[pl.ds(i * 16, 16)] = zero16
        return 0

    lax.fori_loop(0, NP // 16, zh, 0)

    epb = E // NS
    ebase = c * (ERP * CHUNK) + s * epb

    def scat(buf, n):
        for j in range(n // 16):
            v = buf[pl.ds(j * 16, 16)]
            plsc.addupdate_scatter(hist, [v], one16)

    nbig = epb // _BIG
    tail = epb - nbig * _BIG

    def big_pair(q, _):
        offa = pl.multiple_of(ebase + (2 * q) * _BIG, 8)
        offb = pl.multiple_of(ebase + (2 * q + 1) * _BIG, 8)
        da = pltpu.async_copy(ei_ref.at[pl.ds(offa, _BIG)], iva, sia)
        db = pltpu.async_copy(ei_ref.at[pl.ds(offb, _BIG)], ivb, sib)
        da.wait()
        scat(iva, _BIG)
        db.wait()
        scat(ivb, _BIG)
        return 0

    lax.fori_loop(0, nbig // 2, big_pair, 0)
    offa = pl.multiple_of(ebase + (nbig - 1) * _BIG, 8)
    offb = pl.multiple_of(ebase + nbig * _BIG, 8)
    da = pltpu.async_copy(ei_ref.at[pl.ds(offa, _BIG)], iva, sia)
    db = pltpu.async_copy(ei_ref.at[pl.ds(offb, tail)], ivb.at[pl.ds(0, tail)],
                          sib)
    da.wait()
    scat(iva, _BIG)
    db.wait()
    scat(ivb, tail)

    pltpu.sync_copy(hist, shared.at[s])
    plsc.subcore_barrier()
    base = s * ROWS_PER_TILE

    def zsum(g, _):
        sums[pl.ds(g * 16, 16)] = zero16
        return 0

    lax.fori_loop(0, ROWS_PER_TILE // 16, zsum, 0)
    for t in range(NS):
        pltpu.sync_copy(shared.at[t, pl.ds(base, ROWS_PER_TILE)], tmp)

        def addg(g, _):
            sums[pl.ds(g * 16, 16)] = sums[pl.ds(g * 16, 16)] + tmp[pl.ds(g * 16, 16)]
            return 0

        lax.fori_loop(0, ROWS_PER_TILE // 16, addg, 0)

    iota16 = lax.iota(jnp.int32, 16)

    def wg(g, _):
        v = sums[pl.ds(g * 16, 16)]
        plsc.store_scatter(vbuf, [iota16 * 16 + g * 256], v)
        return 0

    lax.fori_loop(0, ROWS_PER_TILE // 16, wg, 0)
    obase = c * (NP * 16) + s * (ROWS_PER_TILE * 16)
    pltpu.sync_copy(vbuf, out_ref.at[pl.ds(obase, ROWS_PER_TILE * 16)])


_sc_degree = functools.partial(
    pl.kernel,
    _deg_body,
    out_type=jax.ShapeDtypeStruct((2 * NP * 16,), jnp.float32),
    mesh=_MESH,
    scratch_types=[
        pltpu.VMEM((_BIG,), jnp.int32),
        pltpu.VMEM((_BIG,), jnp.int32),
        pltpu.VMEM((NP,), jnp.float32),
        pltpu.VMEM((ROWS_PER_TILE,), jnp.float32),
        pltpu.VMEM((ROWS_PER_TILE,), jnp.float32),
        pltpu.VMEM((ROWS_PER_TILE * 16,), jnp.float32),
        pltpu.VMEM_SHARED((NS, NP), jnp.float32),
        pltpu.SemaphoreType.DMA,
        pltpu.SemaphoreType.DMA,
    ],
    compiler_params=pltpu.CompilerParams(needs_layout_passes=False),
)


def _do_block(ei2d, acc, cN, r0, nrows, gather_ref, rows_a, rows_b,
              sidxb, didxb, semi, sga, sgb, ssa, ssb):
    r0 = pl.multiple_of(r0, 8)
    da = pltpu.async_copy(ei2d.at[pl.ds(r0, nrows)], sidxb.at[pl.ds(0, nrows)],
                          semi)
    db = pltpu.async_copy(ei2d.at[pl.ds(r0 + ERP, nrows)],
                          didxb.at[pl.ds(0, nrows)], semi)
    da.wait()
    db.wait()
    if cN is not None:
        for r in range(nrows):
            for j in range(CHUNK // 16):
                v = sidxb[r, pl.ds(j * 16, 16)]
                sidxb[r, pl.ds(j * 16, 16)] = v + cN
    for p in range(nrows // 2):
        ga = pltpu.async_copy(gather_ref.at[sidxb.at[2 * p]], rows_a, sga)
        gb = pltpu.async_copy(gather_ref.at[sidxb.at[2 * p + 1]], rows_b, sgb)
        ga.wait()
        sa = pltpu.async_copy(rows_a, acc.at[didxb.at[2 * p]], ssa, add=True)
        gb.wait()
        sb = pltpu.async_copy(rows_b, acc.at[didxb.at[2 * p + 1]], ssb,
                              add=True)
        sa.wait()
        sb.wait()


def _spmm_scratch():
    return [
        pltpu.VMEM((CHUNK, 128), jnp.float32),
        pltpu.VMEM((CHUNK, 128), jnp.float32),
        pltpu.VMEM((8, CHUNK), jnp.int32),
        pltpu.VMEM((8, CHUNK), jnp.int32),
        pltpu.VMEM_SHARED((NP, 128), jnp.float32),
        pltpu.SemaphoreType.DMA,
        pltpu.SemaphoreType.DMA,
        pltpu.SemaphoreType.DMA,
        pltpu.SemaphoreType.DMA,
        pltpu.SemaphoreType.DMA,
    ]


def _spmm1_body(x2_ref, ei2d_ref, out_ref, rows_a, rows_b, sidxb, didxb,
                acc, semi, sga, sgb, ssa, ssb):
    c = lax.axis_index("c")
    s = lax.axis_index("s")
    _fill_rows(rows_a, CHUNK, 128, 0.0)
    base_row = s * ROWS_PER_TILE
    for r0, nr in _ZERO_SLICES:
        pltpu.sync_copy(rows_a.at[pl.ds(0, nr)], acc.at[pl.ds(base_row + r0, nr)])
    plsc.subcore_barrier()

    cN = c * NP
    args = (x2_ref, rows_a, rows_b, sidxb, didxb, semi, sga, sgb, ssa, ssb)

    def blk(bi, _):
        _do_block(ei2d_ref, acc, cN, s * 80 + bi * 8, 8, *args)
        return 0

    lax.fori_loop(0, 6, blk, 0)

    @pl.when(s < 15)
    def _():
        def blk2(bi, _):
            _do_block(ei2d_ref, acc, cN, s * 80 + bi * 8, 8, *args)
            return 0

        lax.fori_loop(6, 10, blk2, 0)

    @pl.when(s == 15)
    def _():
        _do_block(ei2d_ref, acc, cN, ER - 2, 2, *args)

    plsc.subcore_barrier()
    pltpu.sync_copy(acc.at[pl.ds(base_row, ROWS_PER_TILE)],
                    out_ref.at[pl.ds(cN + base_row, ROWS_PER_TILE)])


_sc_spmm1 = functools.partial(
    pl.kernel,
    _spmm1_body,
    out_type=jax.ShapeDtypeStruct((2 * NP, 128), jnp.float32),
    mesh=_MESH,
    scratch_types=_spmm_scratch(),
)


def _spmm2_body(z_ref, ei2d_ref, out_ref, rows_a, rows_b, sidxb, didxb,
                acc, semi, sga, sgb, ssa, ssb):
    c = lax.axis_index("c")
    s = lax.axis_index("s")
    _fill_rows(rows_a, CHUNK, 128, 0.0)
    base_row = s * ROWS_PER_TILE
    for r0, nr in _ZERO_SLICES:
        pltpu.sync_copy(rows_a.at[pl.ds(0, nr)], acc.at[pl.ds(base_row + r0, nr)])
    plsc.subcore_barrier()

    w = c * NS + s
    args = (z_ref, rows_a, rows_b, sidxb, didxb, semi, sga, sgb, ssa, ssb)

    def blk(bi, _):
        _do_block(ei2d_ref, acc, None, w * 40 + bi * 8, 8, *args)
        return 0

    lax.fori_loop(0, 1, blk, 0)

    @pl.when(w < 31)
    def _():
        def blk2(bi, _):
            _do_block(ei2d_ref, acc, None, w * 40 + bi * 8, 8, *args)
            return 0

        lax.fori_loop(1, 5, blk2, 0)

    @pl.when(w == 31)
    def _():
        _do_block(ei2d_ref, acc, None, ER - 2, 2, *args)

    plsc.subcore_barrier()
    pltpu.sync_copy(acc.at[pl.ds(base_row, ROWS_PER_TILE)],
                    out_ref.at[pl.ds(c * NP + base_row, ROWS_PER_TILE)])


_sc_spmm2 = functools.partial(
    pl.kernel,
    _spmm2_body,
    out_type=jax.ShapeDtypeStruct((2 * NP, 128), jnp.float32),
    mesh=_MESH,
    scratch_types=_spmm_scratch(),
)


BN = 400


def _norms(deg_blk):
    ns = lax.rsqrt(jnp.maximum(deg_blk[0, :, 0:1], 1.0))
    nd = lax.rsqrt(jnp.maximum(deg_blk[1, :, 0:1], 1.0))
    return ns, nd


def _tc_prep_body(deg_ref, x_ref, o_ref):
    ns, _ = _norms(deg_ref[...])
    xn = x_ref[...] * ns
    o_ref[0] = xn[:, :128]
    o_ref[1] = xn[:, 128:]


def _tc_prep(deg2, x):
    return pl.pallas_call(
        _tc_prep_body,
        grid=(N // BN,),
        in_specs=[
            pl.BlockSpec((2, BN, 16), lambda i: (0, i, 0)),
            pl.BlockSpec((BN, D_IN), lambda i: (i, 0)),
        ],
        out_specs=pl.BlockSpec((2, BN, 128), lambda i: (0, i, 0)),
        out_shape=jax.ShapeDtypeStruct((2, NP, 128), jnp.float32),
    )(deg2, x)


def _tc_mid_body(deg_ref, a_ref, w1_ref, b1_ref, w2_ref, z_ref):
    ns, nd = _norms(deg_ref[...])
    a0 = a_ref[0] * nd
    a1 = a_ref[1] * nd
    h = jnp.dot(a0, w1_ref[0:128, :], preferred_element_type=jnp.float32)
    h += jnp.dot(a1, w1_ref[128:256, :], preferred_element_type=jnp.float32)
    h = jnp.maximum(h + b1_ref[...], 0.0)
    z_ref[...] = jnp.dot(h * ns, w2_ref[...], preferred_element_type=jnp.float32)


def _tc_mid(deg2, agg1, W1, b1, W2):
    return pl.pallas_call(
        _tc_mid_body,
        grid=(N // BN,),
        in_specs=[
            pl.BlockSpec((2, BN, 16), lambda i: (0, i, 0)),
            pl.BlockSpec((2, BN, 128), lambda i: (0, i, 0)),
            pl.BlockSpec((D_IN, D_H), lambda i: (0, 0)),
            pl.BlockSpec((1, D_H), lambda i: (0, 0)),
            pl.BlockSpec((D_H, 128), lambda i: (0, 0)),
        ],
        out_specs=pl.BlockSpec((BN, 128), lambda i: (i, 0)),
        out_shape=jax.ShapeDtypeStruct((NP, 128), jnp.float32),
    )(deg2, agg1, W1, b1, W2)


def _tc_fin_body(deg_ref, az_ref, b2_ref, h_ref, p_ref):
    _, nd = _norms(deg_ref[...])
    h = (az_ref[0] + az_ref[1])[:, :D_OUT] * nd + b2_ref[...]
    h_ref[...] = h

    @pl.when(pl.program_id(0) == 0)
    def _():
        p_ref[...] = jnp.zeros_like(p_ref)

    p_ref[...] += jnp.sum(h, axis=0, keepdims=True) * (1.0 / N)


def _tc_fin(deg2, aggz, b2):
    return pl.pallas_call(
        _tc_fin_body,
        grid=(N // BN,),
        in_specs=[
            pl.BlockSpec((2, BN, 16), lambda i: (0, i, 0)),
            pl.BlockSpec((2, BN, 128), lambda i: (0, i, 0)),
            pl.BlockSpec((1, D_OUT), lambda i: (0, 0)),
        ],
        out_specs=[
            pl.BlockSpec((BN, D_OUT), lambda i: (i, 0)),
            pl.BlockSpec((1, D_OUT), lambda i: (0, 0)),
        ],
        out_shape=[
            jax.ShapeDtypeStruct((N, D_OUT), jnp.float32),
            jax.ShapeDtypeStruct((1, D_OUT), jnp.float32),
        ],
    )(deg2, aggz, b2)


def kernel(in_feat, edge_index, W1, b1, W2, b2):
    ei32 = edge_index.astype(jnp.int32)
    pad = jnp.zeros(((ERP - ER) * CHUNK,), jnp.int32)
    eiflat = jnp.concatenate([ei32[0], pad, ei32[1], pad])
    ei2d = eiflat.reshape(2 * ERP, CHUNK)

    deg2 = _sc_degree()(eiflat).reshape(2, NP, 16)
    xs = _tc_prep(deg2, in_feat)
    agg1 = _sc_spmm1()(xs.reshape(2 * NP, 128), ei2d)
    W2p = jnp.pad(W2, ((0, 0), (0, 128 - D_OUT)))
    z = _tc_mid(deg2, agg1.reshape(2, NP, 128), W1, b1.reshape(1, D_H), W2p)
    aggz = _sc_spmm2()(z, ei2d)
    h2, pooled = _tc_fin(deg2, aggz.reshape(2, NP, 128), b2.reshape(1, D_OUT))
    return (h2, pooled)


# speedup vs baseline: 8.5710x; 1.1320x over previous
import functools

import jax
import jax.numpy as jnp
from jax import lax
from jax.experimental import pallas as pl
from jax.experimental.pallas import tpu as pltpu
from jax.experimental.pallas import tpu_sc as plsc

N = 10000
E = 160000
D_IN = 256
D_H = 256
D_OUT = 64

NC = 2
NS = 16
NP = 10240
ROWS_PER_TILE = NP // NS
CHUNK = 128

ER = E // CHUNK
ERP = 1280

_MESH = plsc.VectorSubcoreMesh(core_axis_name="c", subcore_axis_name="s")

_ZERO_SLICES = ((0, 128), (128, 128), (256, 128), (384, 128), (512, 128))


def _fill_rows(ref, nrows, ncols, value):
    vec = jnp.full((16,), value, jnp.float32)

    def body(i, _):
        for j in range(ncols // 16):
            ref[i, pl.ds(j * 16, 16)] = vec
        return 0

    lax.fori_loop(0, nrows, body, 0)


_BIG = 1024


def _deg_body(ei_ref, out_ref, iva, ivb, hist, tmp, sums, vbuf, shared,
              sia, sib):
    c = lax.axis_index("c")
    s = lax.axis_index("s")
    zero16 = jnp.zeros((16,), jnp.float32)
    one16 = jnp.ones((16,), jnp.float32)

    def zh(i, _):
        hist[pl.ds(i * 16, 16)] = zero16
        return 0

    lax.fori_loop(0, NP // 16, zh, 0)

    epb = E // NS
    ebase = c * (ERP * CHUNK) + s * epb

    def scat(buf, n):
        for j in range(n // 16):
            v = buf[pl.ds(j * 16, 16)]
            plsc.addupdate_scatter(hist, [v], one16)

    nbig = epb // _BIG
    tail = epb - nbig * _BIG

    def big_pair(q, _):
        offa = pl.multiple_of(ebase + (2 * q) * _BIG, 8)
        offb = pl.multiple_of(ebase + (2 * q + 1) * _BIG, 8)
        da = pltpu.async_copy(ei_ref.at[pl.ds(offa, _BIG)], iva, sia)
        db = pltpu.async_copy(ei_ref.at[pl.ds(offb, _BIG)], ivb, sib)
        da.wait()
        scat(iva, _BIG)
        db.wait()
        scat(ivb, _BIG)
        return 0

    lax.fori_loop(0, nbig // 2, big_pair, 0)
    offa = pl.multiple_of(ebase + (nbig - 1) * _BIG, 8)
    offb = pl.multiple_of(ebase + nbig * _BIG, 8)
    da = pltpu.async_copy(ei_ref.at[pl.ds(offa, _BIG)], iva, sia)
    db = pltpu.async_copy(ei_ref.at[pl.ds(offb, tail)], ivb.at[pl.ds(0, tail)],
                          sib)
    da.wait()
    scat(iva, _BIG)
    db.wait()
    scat(ivb, tail)

    pltpu.sync_copy(hist, shared.at[s])
    plsc.subcore_barrier()
    base = s * ROWS_PER_TILE

    def zsum(g, _):
        sums[pl.ds(g * 16, 16)] = zero16
        return 0

    lax.fori_loop(0, ROWS_PER_TILE // 16, zsum, 0)
    for t in range(NS):
        pltpu.sync_copy(shared.at[t, pl.ds(base, ROWS_PER_TILE)], tmp)

        def addg(g, _):
            sums[pl.ds(g * 16, 16)] = sums[pl.ds(g * 16, 16)] + tmp[pl.ds(g * 16, 16)]
            return 0

        lax.fori_loop(0, ROWS_PER_TILE // 16, addg, 0)

    iota16 = lax.iota(jnp.int32, 16)

    def wg(g, _):
        v = sums[pl.ds(g * 16, 16)]
        plsc.store_scatter(vbuf, [iota16 * 16 + g * 256], v)
        return 0

    lax.fori_loop(0, ROWS_PER_TILE // 16, wg, 0)
    obase = c * (NP * 16) + s * (ROWS_PER_TILE * 16)
    pltpu.sync_copy(vbuf, out_ref.at[pl.ds(obase, ROWS_PER_TILE * 16)])


_sc_degree = functools.partial(
    pl.kernel,
    _deg_body,
    out_type=jax.ShapeDtypeStruct((2 * NP * 16,), jnp.float32),
    mesh=_MESH,
    scratch_types=[
        pltpu.VMEM((_BIG,), jnp.int32),
        pltpu.VMEM((_BIG,), jnp.int32),
        pltpu.VMEM((NP,), jnp.float32),
        pltpu.VMEM((ROWS_PER_TILE,), jnp.float32),
        pltpu.VMEM((ROWS_PER_TILE,), jnp.float32),
        pltpu.VMEM((ROWS_PER_TILE * 16,), jnp.float32),
        pltpu.VMEM_SHARED((NS, NP), jnp.float32),
        pltpu.SemaphoreType.DMA,
        pltpu.SemaphoreType.DMA,
    ],
    compiler_params=pltpu.CompilerParams(needs_layout_passes=False),
)


def _w_scat(gref, rows, semS, b):
    pltpu.make_async_copy(gref.at[pl.ds(0, CHUNK)], rows[b], semS[b]).wait()


def _block_pipe(ei2d, acc, cN, r0, nrows, gref, rows, sidx, didx,
                semi, semG, semS):
    r0 = pl.multiple_of(r0, 8)
    pltpu.async_copy(ei2d.at[pl.ds(r0, nrows)], sidx.at[pl.ds(0, nrows)],
                     semi)
    pltpu.async_copy(ei2d.at[pl.ds(r0 + ERP, nrows)],
                     didx.at[pl.ds(0, nrows)], semi)
    pltpu.make_async_copy(ei2d.at[pl.ds(0, nrows)],
                          sidx.at[pl.ds(0, nrows)], semi).wait()
    pltpu.make_async_copy(ei2d.at[pl.ds(0, nrows)],
                          didx.at[pl.ds(0, nrows)], semi).wait()
    if cN is not None:
        for r in range(nrows):
            for j in range(CHUNK // 16):
                v = sidx[r, pl.ds(j * 16, 16)]
                sidx[r, pl.ds(j * 16, 16)] = v + cN
    for p in range(nrows):
        b = p % 2
        _w_scat(gref, rows, semS, b)
        pltpu.async_copy(gref.at[sidx.at[p]], rows[b], semG[b])
        if p >= 1:
            b1 = (p - 1) % 2
            pltpu.make_async_copy(gref.at[pl.ds(0, CHUNK)], rows[b1],
                                  semG[b1]).wait()
            pltpu.async_copy(rows[b1], acc.at[didx.at[p - 1]], semS[b1],
                             add=True)
    bl = (nrows - 1) % 2
    pltpu.make_async_copy(gref.at[pl.ds(0, CHUNK)], rows[bl], semG[bl]).wait()
    pltpu.async_copy(rows[bl], acc.at[didx.at[nrows - 1]], semS[bl], add=True)


def _prime_scatters(acc, rows, didx1, semS):
    pltpu.async_copy(rows[0], acc.at[didx1.at[0]], semS[0], add=True)
    pltpu.async_copy(rows[1], acc.at[didx1.at[1]], semS[1], add=True)


def _spmm_scratch():
    return [
        pltpu.VMEM((CHUNK, 128), jnp.float32),
        pltpu.VMEM((CHUNK, 128), jnp.float32),
        pltpu.VMEM((8, CHUNK), jnp.int32),
        pltpu.VMEM((8, CHUNK), jnp.int32),
        pltpu.VMEM((8, CHUNK), jnp.int32),
        pltpu.VMEM((8, CHUNK), jnp.int32),
        pltpu.VMEM_SHARED((NP, 128), jnp.float32),
        pltpu.SemaphoreType.DMA,
        pltpu.SemaphoreType.DMA,
        pltpu.SemaphoreType.DMA,
        pltpu.SemaphoreType.DMA,
        pltpu.SemaphoreType.DMA,
    ]


def _spmm_init(rows_a, rows_b, didx1, acc, base_row):
    _fill_rows(rows_a, CHUNK, 128, 0.0)
    _fill_rows(rows_b, CHUNK, 128, 0.0)
    zero16i = jnp.zeros((16,), jnp.int32)
    for r in range(2):
        for j in range(CHUNK // 16):
            didx1[r, pl.ds(j * 16, 16)] = zero16i
    for r0, nr in _ZERO_SLICES:
        pltpu.sync_copy(rows_a.at[pl.ds(0, nr)],
                        acc.at[pl.ds(base_row + r0, nr)])


def _spmm1_body(x2_ref, ei2d_ref, out_ref, rows_a, rows_b, sidx0, didx0,
                sidx1, didx1, acc, semi, sga, sgb, ssa, ssb):
    c = lax.axis_index("c")
    s = lax.axis_index("s")
    base_row = s * ROWS_PER_TILE
    _spmm_init(rows_a, rows_b, didx1, acc, base_row)
    plsc.subcore_barrier()

    cN = c * NP
    rows = (rows_a, rows_b)
    semG = (sga, sgb)
    semS = (ssa, ssb)
    _prime_scatters(acc, rows, didx1, semS)

    def run(r0, nrows, sidx, didx):
        _block_pipe(ei2d_ref, acc, cN, r0, nrows, x2_ref, rows, sidx, didx,
                    semi, semG, semS)

    def pairq(q, _):
        run(s * 80 + (2 * q) * 8, 8, sidx0, didx0)
        run(s * 80 + (2 * q + 1) * 8, 8, sidx1, didx1)
        return 0

    lax.fori_loop(0, 3, pairq, 0)

    @pl.when(s < 15)
    def _():
        def pairq2(q, _):
            run(s * 80 + (2 * q) * 8, 8, sidx0, didx0)
            run(s * 80 + (2 * q + 1) * 8, 8, sidx1, didx1)
            return 0

        lax.fori_loop(3, 5, pairq2, 0)

    @pl.when(s == 15)
    def _():
        run(ER - 2, 2, sidx1, didx1)

    _w_scat(x2_ref, rows, semS, 0)
    _w_scat(x2_ref, rows, semS, 1)
    plsc.subcore_barrier()
    pltpu.sync_copy(acc.at[pl.ds(base_row, ROWS_PER_TILE)],
                    out_ref.at[pl.ds(cN + base_row, ROWS_PER_TILE)])


_sc_spmm1 = functools.partial(
    pl.kernel,
    _spmm1_body,
    out_type=jax.ShapeDtypeStruct((2 * NP, 128), jnp.float32),
    mesh=_MESH,
    scratch_types=_spmm_scratch(),
)


def _spmm2_body(z_ref, ei2d_ref, out_ref, rows_a, rows_b, sidx0, didx0,
                sidx1, didx1, acc, semi, sga, sgb, ssa, ssb):
    c = lax.axis_index("c")
    s = lax.axis_index("s")
    base_row = s * ROWS_PER_TILE
    _spmm_init(rows_a, rows_b, didx1, acc, base_row)
    plsc.subcore_barrier()

    w = c * NS + s
    rows = (rows_a, rows_b)
    semG = (sga, sgb)
    semS = (ssa, ssb)
    _prime_scatters(acc, rows, didx1, semS)

    def run(r0, nrows, sidx, didx):
        _block_pipe(ei2d_ref, acc, None, r0, nrows, z_ref, rows, sidx, didx,
                    semi, semG, semS)

    run(w * 40, 8, sidx0, didx0)

    @pl.when(w < 31)
    def _():
        def pairq(q, _):
            run(w * 40 + (2 * q + 1) * 8, 8, sidx1, didx1)
            run(w * 40 + (2 * q + 2) * 8, 8, sidx0, didx0)
            return 0

        lax.fori_loop(0, 2, pairq, 0)

    @pl.when(w == 31)
    def _():
        run(ER - 2, 2, sidx1, didx1)

    _w_scat(z_ref, rows, semS, 0)
    _w_scat(z_ref, rows, semS, 1)
    plsc.subcore_barrier()
    pltpu.sync_copy(acc.at[pl.ds(base_row, ROWS_PER_TILE)],
                    out_ref.at[pl.ds(c * NP + base_row, ROWS_PER_TILE)])


_sc_spmm2 = functools.partial(
    pl.kernel,
    _spmm2_body,
    out_type=jax.ShapeDtypeStruct((2 * NP, 128), jnp.float32),
    mesh=_MESH,
    scratch_types=_spmm_scratch(),
)


BN = 400


def _norms(deg_blk):
    ns = lax.rsqrt(jnp.maximum(deg_blk[0, :, 0:1], 1.0))
    nd = lax.rsqrt(jnp.maximum(deg_blk[1, :, 0:1], 1.0))
    return ns, nd


def _tc_prep_body(deg_ref, x_ref, o_ref):
    ns, _ = _norms(deg_ref[...])
    xn = x_ref[...] * ns
    o_ref[0] = xn[:, :128]
    o_ref[1] = xn[:, 128:]


def _tc_prep(deg2, x):
    return pl.pallas_call(
        _tc_prep_body,
        grid=(N // BN,),
        in_specs=[
            pl.BlockSpec((2, BN, 16), lambda i: (0, i, 0)),
            pl.BlockSpec((BN, D_IN), lambda i: (i, 0)),
        ],
        out_specs=pl.BlockSpec((2, BN, 128), lambda i: (0, i, 0)),
        out_shape=jax.ShapeDtypeStruct((2, NP, 128), jnp.float32),
    )(deg2, x)


def _tc_mid_body(deg_ref, a_ref, w1_ref, b1_ref, w2_ref, z_ref):
    ns, nd = _norms(deg_ref[...])
    a0 = a_ref[0] * nd
    a1 = a_ref[1] * nd
    h = jnp.dot(a0, w1_ref[0:128, :], preferred_element_type=jnp.float32)
    h += jnp.dot(a1, w1_ref[128:256, :], preferred_element_type=jnp.float32)
    h = jnp.maximum(h + b1_ref[...], 0.0)
    z_ref[...] = jnp.dot(h * ns, w2_ref[...], preferred_element_type=jnp.float32)


def _tc_mid(deg2, agg1, W1, b1, W2):
    return pl.pallas_call(
        _tc_mid_body,
        grid=(N // BN,),
        in_specs=[
            pl.BlockSpec((2, BN, 16), lambda i: (0, i, 0)),
            pl.BlockSpec((2, BN, 128), lambda i: (0, i, 0)),
            pl.BlockSpec((D_IN, D_H), lambda i: (0, 0)),
            pl.BlockSpec((1, D_H), lambda i: (0, 0)),
            pl.BlockSpec((D_H, 128), lambda i: (0, 0)),
        ],
        out_specs=pl.BlockSpec((BN, 128), lambda i: (i, 0)),
        out_shape=jax.ShapeDtypeStruct((NP, 128), jnp.float32),
    )(deg2, agg1, W1, b1, W2)


def _tc_fin_body(deg_ref, az_ref, b2_ref, h_ref, p_ref):
    _, nd = _norms(deg_ref[...])
    h = (az_ref[0] + az_ref[1])[:, :D_OUT] * nd + b2_ref[...]
    h_ref[...] = h

    @pl.when(pl.program_id(0) == 0)
    def _():
        p_ref[...] = jnp.zeros_like(p_ref)

    p_ref[...] += jnp.sum(h, axis=0, keepdims=True) * (1.0 / N)


def _tc_fin(deg2, aggz, b2):
    return pl.pallas_call(
        _tc_fin_body,
        grid=(N // BN,),
        in_specs=[
            pl.BlockSpec((2, BN, 16), lambda i: (0, i, 0)),
            pl.BlockSpec((2, BN, 128), lambda i: (0, i, 0)),
            pl.BlockSpec((1, D_OUT), lambda i: (0, 0)),
        ],
        out_specs=[
            pl.BlockSpec((BN, D_OUT), lambda i: (i, 0)),
            pl.BlockSpec((1, D_OUT), lambda i: (0, 0)),
        ],
        out_shape=[
            jax.ShapeDtypeStruct((N, D_OUT), jnp.float32),
            jax.ShapeDtypeStruct((1, D_OUT), jnp.float32),
        ],
    )(deg2, aggz, b2)


def kernel(in_feat, edge_index, W1, b1, W2, b2):
    ei32 = edge_index.astype(jnp.int32)
    pad = jnp.zeros(((ERP - ER) * CHUNK,), jnp.int32)
    eiflat = jnp.concatenate([ei32[0], pad, ei32[1], pad])
    ei2d = eiflat.reshape(2 * ERP, CHUNK)

    deg2 = _sc_degree()(eiflat).reshape(2, NP, 16)
    xs = _tc_prep(deg2, in_feat)
    agg1 = _sc_spmm1()(xs.reshape(2 * NP, 128), ei2d)
    W2p = jnp.pad(W2, ((0, 0), (0, 128 - D_OUT)))
    z = _tc_mid(deg2, agg1.reshape(2, NP, 128), W1, b1.reshape(1, D_H), W2p)
    aggz = _sc_spmm2()(z, ei2d)
    h2, pooled = _tc_fin(deg2, aggz.reshape(2, NP, 128), b2.reshape(1, D_OUT))
    return (h2, pooled)


# speedup vs baseline: 9.3168x; 1.0870x over previous
import functools

import jax
import jax.numpy as jnp
from jax import lax
from jax.experimental import pallas as pl
from jax.experimental.pallas import tpu as pltpu
from jax.experimental.pallas import tpu_sc as plsc

N = 10000
E = 160000
D_IN = 256
D_H = 256
D_OUT = 64

NC = 2
NS = 16
NP = 10240
ROWS_PER_TILE = NP // NS
CHUNK = 128

ER = E // CHUNK
ERP = 1280

_MESH = plsc.VectorSubcoreMesh(core_axis_name="c", subcore_axis_name="s")

_ZERO_SLICES = ((0, 128), (128, 128), (256, 128), (384, 128), (512, 128))


def _fill_rows(ref, nrows, ncols, value):
    vec = jnp.full((16,), value, jnp.float32)

    def body(i, _):
        for j in range(ncols // 16):
            ref[i, pl.ds(j * 16, 16)] = vec
        return 0

    lax.fori_loop(0, nrows, body, 0)


_BIG = 1024


def _deg_body(ei_ref, out_ref, iva, ivb, hist, tmp, sums, vbuf, shared,
              sia, sib):
    c = lax.axis_index("c")
    s = lax.axis_index("s")
    zero16 = jnp.zeros((16,), jnp.float32)
    one16 = jnp.ones((16,), jnp.float32)

    def zh(i, _):
        hist[pl.ds(i * 16, 16)] = zero16
        return 0

    lax.fori_loop(0, NP // 16, zh, 0)

    epb = E // NS
    ebase = c * (ERP * CHUNK) + s * epb

    def scat(buf, n):
        for j in range(n // 16):
            v = buf[pl.ds(j * 16, 16)]
            plsc.addupdate_scatter(hist, [v], one16)

    nbig = epb // _BIG
    tail = epb - nbig * _BIG

    def big_pair(q, _):
        offa = pl.multiple_of(ebase + (2 * q) * _BIG, 8)
        offb = pl.multiple_of(ebase + (2 * q + 1) * _BIG, 8)
        da = pltpu.async_copy(ei_ref.at[pl.ds(offa, _BIG)], iva, sia)
        db = pltpu.async_copy(ei_ref.at[pl.ds(offb, _BIG)], ivb, sib)
        da.wait()
        scat(iva, _BIG)
        db.wait()
        scat(ivb, _BIG)
        return 0

    lax.fori_loop(0, nbig // 2, big_pair, 0)
    offa = pl.multiple_of(ebase + (nbig - 1) * _BIG, 8)
    offb = pl.multiple_of(ebase + nbig * _BIG, 8)
    da = pltpu.async_copy(ei_ref.at[pl.ds(offa, _BIG)], iva, sia)
    db = pltpu.async_copy(ei_ref.at[pl.ds(offb, tail)], ivb.at[pl.ds(0, tail)],
                          sib)
    da.wait()
    scat(iva, _BIG)
    db.wait()
    scat(ivb, tail)

    pltpu.sync_copy(hist, shared.at[s])
    plsc.subcore_barrier()
    base = s * ROWS_PER_TILE

    def zsum(g, _):
        sums[pl.ds(g * 16, 16)] = zero16
        return 0

    lax.fori_loop(0, ROWS_PER_TILE // 16, zsum, 0)
    for t in range(NS):
        pltpu.sync_copy(shared.at[t, pl.ds(base, ROWS_PER_TILE)], tmp)

        def addg(g, _):
            sums[pl.ds(g * 16, 16)] = sums[pl.ds(g * 16, 16)] + tmp[pl.ds(g * 16, 16)]
            return 0

        lax.fori_loop(0, ROWS_PER_TILE // 16, addg, 0)

    iota16 = lax.iota(jnp.int32, 16)

    def wg(g, _):
        v = sums[pl.ds(g * 16, 16)]
        plsc.store_scatter(vbuf, [iota16 * 16 + g * 256], v)
        return 0

    lax.fori_loop(0, ROWS_PER_TILE // 16, wg, 0)
    obase = c * (NP * 16) + s * (ROWS_PER_TILE * 16)
    pltpu.sync_copy(vbuf, out_ref.at[pl.ds(obase, ROWS_PER_TILE * 16)])


_sc_degree = functools.partial(
    pl.kernel,
    _deg_body,
    out_type=jax.ShapeDtypeStruct((2 * NP * 16,), jnp.float32),
    mesh=_MESH,
    scratch_types=[
        pltpu.VMEM((_BIG,), jnp.int32),
        pltpu.VMEM((_BIG,), jnp.int32),
        pltpu.VMEM((NP,), jnp.float32),
        pltpu.VMEM((ROWS_PER_TILE,), jnp.float32),
        pltpu.VMEM((ROWS_PER_TILE,), jnp.float32),
        pltpu.VMEM((ROWS_PER_TILE * 16,), jnp.float32),
        pltpu.VMEM_SHARED((NS, NP), jnp.float32),
        pltpu.SemaphoreType.DMA,
        pltpu.SemaphoreType.DMA,
    ],
    compiler_params=pltpu.CompilerParams(needs_layout_passes=False),
)


def _w_scat(gref, rows, semS, b):
    pltpu.make_async_copy(gref.at[pl.ds(0, CHUNK)], rows[b], semS[b]).wait()


def _block_pipe(ei2d, acc, cN, r0, nrows, gref, rows, sidx, didx,
                semi, semG, semS):
    r0 = pl.multiple_of(r0, 8)
    pltpu.async_copy(ei2d.at[pl.ds(r0, nrows)], sidx.at[pl.ds(0, nrows)],
                     semi)
    pltpu.async_copy(ei2d.at[pl.ds(r0 + ERP, nrows)],
                     didx.at[pl.ds(0, nrows)], semi)
    pltpu.make_async_copy(ei2d.at[pl.ds(0, nrows)],
                          sidx.at[pl.ds(0, nrows)], semi).wait()
    pltpu.make_async_copy(ei2d.at[pl.ds(0, nrows)],
                          didx.at[pl.ds(0, nrows)], semi).wait()
    if cN is not None:
        for r in range(nrows):
            for j in range(CHUNK // 16):
                v = sidx[r, pl.ds(j * 16, 16)]
                sidx[r, pl.ds(j * 16, 16)] = v + cN
    for p in range(nrows):
        b = p % 2
        _w_scat(gref, rows, semS, b)
        pltpu.async_copy(gref.at[sidx.at[p]], rows[b], semG[b])
        if p >= 1:
            b1 = (p - 1) % 2
            pltpu.make_async_copy(gref.at[pl.ds(0, CHUNK)], rows[b1],
                                  semG[b1]).wait()
            pltpu.async_copy(rows[b1], acc.at[didx.at[p - 1]], semS[b1],
                             add=True)
    bl = (nrows - 1) % 2
    pltpu.make_async_copy(gref.at[pl.ds(0, CHUNK)], rows[bl], semG[bl]).wait()
    pltpu.async_copy(rows[bl], acc.at[didx.at[nrows - 1]], semS[bl], add=True)


def _prime_scatters(acc, rows, didx1, semS):
    pltpu.async_copy(rows[0], acc.at[didx1.at[0]], semS[0], add=True)
    pltpu.async_copy(rows[1], acc.at[didx1.at[1]], semS[1], add=True)


def _spmm_scratch():
    return [
        pltpu.VMEM((CHUNK, 128), jnp.float32),
        pltpu.VMEM((CHUNK, 128), jnp.float32),
        pltpu.VMEM((8, CHUNK), jnp.int32),
        pltpu.VMEM((8, CHUNK), jnp.int32),
        pltpu.VMEM((8, CHUNK), jnp.int32),
        pltpu.VMEM((8, CHUNK), jnp.int32),
        pltpu.VMEM_SHARED((NP, 128), jnp.float32),
        pltpu.SemaphoreType.DMA,
        pltpu.SemaphoreType.DMA,
        pltpu.SemaphoreType.DMA,
        pltpu.SemaphoreType.DMA,
        pltpu.SemaphoreType.DMA,
    ]


def _spmm_init(rows_a, rows_b, didx1, acc, base_row):
    _fill_rows(rows_a, CHUNK, 128, 0.0)
    _fill_rows(rows_b, CHUNK, 128, 0.0)
    zero16i = jnp.zeros((16,), jnp.int32)
    for r in range(2):
        for j in range(CHUNK // 16):
            didx1[r, pl.ds(j * 16, 16)] = zero16i
    for r0, nr in _ZERO_SLICES:
        pltpu.sync_copy(rows_a.at[pl.ds(0, nr)],
                        acc.at[pl.ds(base_row + r0, nr)])


def _spmm1_body(x2_ref, ei2d_ref, out_ref, rows_a, rows_b, sidx0, didx0,
                sidx1, didx1, acc, semi, sga, sgb, ssa, ssb):
    c = lax.axis_index("c")
    s = lax.axis_index("s")
    base_row = s * ROWS_PER_TILE
    _spmm_init(rows_a, rows_b, didx1, acc, base_row)
    plsc.subcore_barrier()

    cN = c * NP
    rows = (rows_a, rows_b)
    semG = (sga, sgb)
    semS = (ssa, ssb)
    _prime_scatters(acc, rows, didx1, semS)

    def run(r0, nrows, sidx, didx):
        _block_pipe(ei2d_ref, acc, cN, r0, nrows, x2_ref, rows, sidx, didx,
                    semi, semG, semS)

    def pairq(q, _):
        run(s * 80 + (2 * q) * 8, 8, sidx0, didx0)
        run(s * 80 + (2 * q + 1) * 8, 8, sidx1, didx1)
        return 0

    lax.fori_loop(0, 3, pairq, 0)

    @pl.when(s < 15)
    def _():
        def pairq2(q, _):
            run(s * 80 + (2 * q) * 8, 8, sidx0, didx0)
            run(s * 80 + (2 * q + 1) * 8, 8, sidx1, didx1)
            return 0

        lax.fori_loop(3, 5, pairq2, 0)

    @pl.when(s == 15)
    def _():
        run(ER - 2, 2, sidx1, didx1)

    _w_scat(x2_ref, rows, semS, 0)
    _w_scat(x2_ref, rows, semS, 1)
    plsc.subcore_barrier()
    pltpu.sync_copy(acc.at[pl.ds(base_row, ROWS_PER_TILE)],
                    out_ref.at[pl.ds(cN + base_row, ROWS_PER_TILE)])


_sc_spmm1 = functools.partial(
    pl.kernel,
    _spmm1_body,
    out_type=jax.ShapeDtypeStruct((2 * NP, 128), jnp.float32),
    mesh=_MESH,
    scratch_types=_spmm_scratch(),
)


def _spmm2_body(z_ref, ei2d_ref, out_ref, rows_a, rows_b, sidx0, didx0,
                sidx1, didx1, acc, semi, sga, sgb, ssa, ssb):
    c = lax.axis_index("c")
    s = lax.axis_index("s")
    base_row = s * ROWS_PER_TILE
    _spmm_init(rows_a, rows_b, didx1, acc, base_row)
    plsc.subcore_barrier()

    w = c * NS + s
    rows = (rows_a, rows_b)
    semG = (sga, sgb)
    semS = (ssa, ssb)
    _prime_scatters(acc, rows, didx1, semS)

    def run(r0, nrows, sidx, didx):
        _block_pipe(ei2d_ref, acc, None, r0, nrows, z_ref, rows, sidx, didx,
                    semi, semG, semS)

    run(w * 40, 8, sidx0, didx0)

    @pl.when(w < 31)
    def _():
        def pairq(q, _):
            run(w * 40 + (2 * q + 1) * 8, 8, sidx1, didx1)
            run(w * 40 + (2 * q + 2) * 8, 8, sidx0, didx0)
            return 0

        lax.fori_loop(0, 2, pairq, 0)

    @pl.when(w == 31)
    def _():
        run(ER - 2, 2, sidx1, didx1)

    _w_scat(z_ref, rows, semS, 0)
    _w_scat(z_ref, rows, semS, 1)
    plsc.subcore_barrier()
    pltpu.sync_copy(acc.at[pl.ds(base_row, ROWS_PER_TILE)],
                    out_ref.at[pl.ds(c * NP + base_row, ROWS_PER_TILE)])


_sc_spmm2 = functools.partial(
    pl.kernel,
    _spmm2_body,
    out_type=jax.ShapeDtypeStruct((2 * NP, 128), jnp.float32),
    mesh=_MESH,
    scratch_types=_spmm_scratch(),
)


BN = 1000


def _norms(deg_blk):
    ns = lax.rsqrt(jnp.maximum(deg_blk[0, :, 0:1], 1.0))
    nd = lax.rsqrt(jnp.maximum(deg_blk[1, :, 0:1], 1.0))
    return ns, nd


def _tc_prep_body(deg_ref, x_ref, o_ref):
    ns, _ = _norms(deg_ref[...])
    xn = x_ref[...] * ns
    o_ref[0] = xn[:, :128]
    o_ref[1] = xn[:, 128:]


def _tc_prep(deg2, x):
    return pl.pallas_call(
        _tc_prep_body,
        grid=(N // BN,),
        in_specs=[
            pl.BlockSpec((2, BN, 16), lambda i: (0, i, 0)),
            pl.BlockSpec((BN, D_IN), lambda i: (i, 0)),
        ],
        out_specs=pl.BlockSpec((2, BN, 128), lambda i: (0, i, 0)),
        out_shape=jax.ShapeDtypeStruct((2, NP, 128), jnp.float32),
    )(deg2, x)


def _tc_mid_body(deg_ref, a_ref, w1_ref, b1_ref, w2_ref, z_ref):
    ns, nd = _norms(deg_ref[...])
    a0 = a_ref[0] * nd
    a1 = a_ref[1] * nd
    h = jnp.dot(a0, w1_ref[0:128, :], preferred_element_type=jnp.float32)
    h += jnp.dot(a1, w1_ref[128:256, :], preferred_element_type=jnp.float32)
    h = jnp.maximum(h + b1_ref[...], 0.0)
    z_ref[...] = jnp.dot(h * ns, w2_ref[...], preferred_element_type=jnp.float32)


def _tc_mid(deg2, agg1, W1, b1, W2):
    return pl.pallas_call(
        _tc_mid_body,
        grid=(N // BN,),
        in_specs=[
            pl.BlockSpec((2, BN, 16), lambda i: (0, i, 0)),
            pl.BlockSpec((2, BN, 128), lambda i: (0, i, 0)),
            pl.BlockSpec((D_IN, D_H), lambda i: (0, 0)),
            pl.BlockSpec((1, D_H), lambda i: (0, 0)),
            pl.BlockSpec((D_H, 128), lambda i: (0, 0)),
        ],
        out_specs=pl.BlockSpec((BN, 128), lambda i: (i, 0)),
        out_shape=jax.ShapeDtypeStruct((NP, 128), jnp.float32),
    )(deg2, agg1, W1, b1, W2)


def _tc_fin_body(deg_ref, az_ref, b2_ref, h_ref, p_ref):
    _, nd = _norms(deg_ref[...])
    h = (az_ref[0] + az_ref[1])[:, :D_OUT] * nd + b2_ref[...]
    h_ref[...] = h

    @pl.when(pl.program_id(0) == 0)
    def _():
        p_ref[...] = jnp.zeros_like(p_ref)

    p_ref[...] += jnp.sum(h, axis=0, keepdims=True) * (1.0 / N)


def _tc_fin(deg2, aggz, b2):
    return pl.pallas_call(
        _tc_fin_body,
        grid=(N // BN,),
        in_specs=[
            pl.BlockSpec((2, BN, 16), lambda i: (0, i, 0)),
            pl.BlockSpec((2, BN, 128), lambda i: (0, i, 0)),
            pl.BlockSpec((1, D_OUT), lambda i: (0, 0)),
        ],
        out_specs=[
            pl.BlockSpec((BN, D_OUT), lambda i: (i, 0)),
            pl.BlockSpec((1, D_OUT), lambda i: (0, 0)),
        ],
        out_shape=[
            jax.ShapeDtypeStruct((N, D_OUT), jnp.float32),
            jax.ShapeDtypeStruct((1, D_OUT), jnp.float32),
        ],
    )(deg2, aggz, b2)


def kernel(in_feat, edge_index, W1, b1, W2, b2):
    ei32 = edge_index.astype(jnp.int32)
    pad = jnp.zeros(((ERP - ER) * CHUNK,), jnp.int32)
    eiflat = jnp.concatenate([ei32[0], pad, ei32[1], pad])
    ei2d = eiflat.reshape(2 * ERP, CHUNK)

    deg2 = _sc_degree()(eiflat).reshape(2, NP, 16)
    xs = _tc_prep(deg2, in_feat)
    agg1 = _sc_spmm1()(xs.reshape(2 * NP, 128), ei2d)
    W2p = jnp.pad(W2, ((0, 0), (0, 128 - D_OUT)))
    z = _tc_mid(deg2, agg1.reshape(2, NP, 128), W1, b1.reshape(1, D_H), W2p)
    aggz = _sc_spmm2()(z, ei2d)
    h2, pooled = _tc_fin(deg2, aggz.reshape(2, NP, 128), b2.reshape(1, D_OUT))
    return (h2, pooled)


# speedup vs baseline: 9.5236x; 1.0222x over previous
import functools

import jax
import jax.numpy as jnp
from jax import lax
from jax.experimental import pallas as pl
from jax.experimental.pallas import tpu as pltpu
from jax.experimental.pallas import tpu_sc as plsc

N = 10000
E = 160000
D_IN = 256
D_H = 256
D_OUT = 64

NC = 2
NS = 16
NP = 10240
ROWS_PER_TILE = NP // NS
CHUNK = 128

ER = E // CHUNK
ERP = 1280

_MESH = plsc.VectorSubcoreMesh(core_axis_name="c", subcore_axis_name="s")

_ZERO_SLICES = ((0, 128), (128, 128), (256, 128), (384, 128), (512, 128))


def _fill_rows(ref, nrows, ncols, value):
    vec = jnp.full((16,), value, jnp.float32)

    def body(i, _):
        for j in range(ncols // 16):
            ref[i, pl.ds(j * 16, 16)] = vec
        return 0

    lax.fori_loop(0, nrows, body, 0)


_BIG = 1024


def _deg_body(ei_ref, out_ref, iva, ivb, hist, tmp, sums, vbuf, shared,
              sia, sib):
    c = lax.axis_index("c")
    s = lax.axis_index("s")
    zero16 = jnp.zeros((16,), jnp.float32)
    one16 = jnp.ones((16,), jnp.float32)

    def zh(i, _):
        hist[pl.ds(i * 16, 16)] = zero16
        return 0

    lax.fori_loop(0, NP // 16, zh, 0)

    epb = E // NS
    ebase = c * (ERP * CHUNK) + s * epb

    def scat(buf, n):
        for j in range(n // 16):
            v = buf[pl.ds(j * 16, 16)]
            plsc.addupdate_scatter(hist, [v], one16)

    nbig = epb // _BIG
    tail = epb - nbig * _BIG

    def big_pair(q, _):
        offa = pl.multiple_of(ebase + (2 * q) * _BIG, 8)
        offb = pl.multiple_of(ebase + (2 * q + 1) * _BIG, 8)
        da = pltpu.async_copy(ei_ref.at[pl.ds(offa, _BIG)], iva, sia)
        db = pltpu.async_copy(ei_ref.at[pl.ds(offb, _BIG)], ivb, sib)
        da.wait()
        scat(iva, _BIG)
        db.wait()
        scat(ivb, _BIG)
        return 0

    lax.fori_loop(0, nbig // 2, big_pair, 0)
    offa = pl.multiple_of(ebase + (nbig - 1) * _BIG, 8)
    offb = pl.multiple_of(ebase + nbig * _BIG, 8)
    da = pltpu.async_copy(ei_ref.at[pl.ds(offa, _BIG)], iva, sia)
    db = pltpu.async_copy(ei_ref.at[pl.ds(offb, tail)], ivb.at[pl.ds(0, tail)],
                          sib)
    da.wait()
    scat(iva, _BIG)
    db.wait()
    scat(ivb, tail)

    pltpu.sync_copy(hist, shared.at[s])
    plsc.subcore_barrier()
    base = s * ROWS_PER_TILE

    def zsum(g, _):
        sums[pl.ds(g * 16, 16)] = zero16
        return 0

    lax.fori_loop(0, ROWS_PER_TILE // 16, zsum, 0)
    for t in range(NS):
        pltpu.sync_copy(shared.at[t, pl.ds(base, ROWS_PER_TILE)], tmp)

        def addg(g, _):
            sums[pl.ds(g * 16, 16)] = sums[pl.ds(g * 16, 16)] + tmp[pl.ds(g * 16, 16)]
            return 0

        lax.fori_loop(0, ROWS_PER_TILE // 16, addg, 0)

    iota16 = lax.iota(jnp.int32, 16)

    def wg(g, _):
        v = sums[pl.ds(g * 16, 16)]
        plsc.store_scatter(vbuf, [iota16 * 16 + g * 256], v)
        return 0

    lax.fori_loop(0, ROWS_PER_TILE // 16, wg, 0)
    obase = c * (NP * 16) + s * (ROWS_PER_TILE * 16)
    pltpu.sync_copy(vbuf, out_ref.at[pl.ds(obase, ROWS_PER_TILE * 16)])


_sc_degree = functools.partial(
    pl.kernel,
    _deg_body,
    out_type=jax.ShapeDtypeStruct((2 * NP * 16,), jnp.float32),
    mesh=_MESH,
    scratch_types=[
        pltpu.VMEM((_BIG,), jnp.int32),
        pltpu.VMEM((_BIG,), jnp.int32),
        pltpu.VMEM((NP,), jnp.float32),
        pltpu.VMEM((ROWS_PER_TILE,), jnp.float32),
        pltpu.VMEM((ROWS_PER_TILE,), jnp.float32),
        pltpu.VMEM((ROWS_PER_TILE * 16,), jnp.float32),
        pltpu.VMEM_SHARED((NS, NP), jnp.float32),
        pltpu.SemaphoreType.DMA,
        pltpu.SemaphoreType.DMA,
    ],
    compiler_params=pltpu.CompilerParams(needs_layout_passes=False),
)


def _w_scat(gref, rows, semS, b):
    pltpu.make_async_copy(gref.at[pl.ds(0, CHUNK)], rows[b], semS[b]).wait()


def _block_pipe(ei2d, acc, cN, r0, nrows, gref, rows, sidx, didx,
                semi, semG, semS):
    r0 = pl.multiple_of(r0, 8)
    pltpu.async_copy(ei2d.at[pl.ds(r0, nrows)], sidx.at[pl.ds(0, nrows)],
                     semi)
    pltpu.async_copy(ei2d.at[pl.ds(r0 + ERP, nrows)],
                     didx.at[pl.ds(0, nrows)], semi)
    pltpu.make_async_copy(ei2d.at[pl.ds(0, nrows)],
                          sidx.at[pl.ds(0, nrows)], semi).wait()
    pltpu.make_async_copy(ei2d.at[pl.ds(0, nrows)],
                          didx.at[pl.ds(0, nrows)], semi).wait()
    if cN is not None:
        for r in range(nrows):
            for j in range(CHUNK // 16):
                v = sidx[r, pl.ds(j * 16, 16)]
                sidx[r, pl.ds(j * 16, 16)] = v + cN
    for p in range(nrows):
        b = p % 2
        _w_scat(gref, rows, semS, b)
        pltpu.async_copy(gref.at[sidx.at[p]], rows[b], semG[b])
        if p >= 1:
            b1 = (p - 1) % 2
            pltpu.make_async_copy(gref.at[pl.ds(0, CHUNK)], rows[b1],
                                  semG[b1]).wait()
            pltpu.async_copy(rows[b1], acc.at[didx.at[p - 1]], semS[b1],
                             add=True)
    bl = (nrows - 1) % 2
    pltpu.make_async_copy(gref.at[pl.ds(0, CHUNK)], rows[bl], semG[bl]).wait()
    pltpu.async_copy(rows[bl], acc.at[didx.at[nrows - 1]], semS[bl], add=True)


def _prime_scatters(acc, rows, didx1, semS):
    pltpu.async_copy(rows[0], acc.at[didx1.at[0]], semS[0], add=True)
    pltpu.async_copy(rows[1], acc.at[didx1.at[1]], semS[1], add=True)


def _spmm_scratch():
    return [
        pltpu.VMEM((CHUNK, 128), jnp.float32),
        pltpu.VMEM((CHUNK, 128), jnp.float32),
        pltpu.VMEM((8, CHUNK), jnp.int32),
        pltpu.VMEM((8, CHUNK), jnp.int32),
        pltpu.VMEM((8, CHUNK), jnp.int32),
        pltpu.VMEM((8, CHUNK), jnp.int32),
        pltpu.VMEM_SHARED((NP, 128), jnp.float32),
        pltpu.SemaphoreType.DMA,
        pltpu.SemaphoreType.DMA,
        pltpu.SemaphoreType.DMA,
        pltpu.SemaphoreType.DMA,
        pltpu.SemaphoreType.DMA,
    ]


def _spmm_init(rows_a, rows_b, didx1, acc, base_row):
    _fill_rows(rows_a, CHUNK, 128, 0.0)
    _fill_rows(rows_b, CHUNK, 128, 0.0)
    zero16i = jnp.zeros((16,), jnp.int32)
    for r in range(2):
        for j in range(CHUNK // 16):
            didx1[r, pl.ds(j * 16, 16)] = zero16i
    for r0, nr in _ZERO_SLICES:
        pltpu.sync_copy(rows_a.at[pl.ds(0, nr)],
                        acc.at[pl.ds(base_row + r0, nr)])


def _spmm1_body(x2_ref, ei2d_ref, out_ref, rows_a, rows_b, sidx0, didx0,
                sidx1, didx1, acc, semi, sga, sgb, ssa, ssb):
    c = lax.axis_index("c")
    s = lax.axis_index("s")
    base_row = s * ROWS_PER_TILE
    _spmm_init(rows_a, rows_b, didx1, acc, base_row)
    plsc.subcore_barrier()

    cN = c * NP
    rows = (rows_a, rows_b)
    semG = (sga, sgb)
    semS = (ssa, ssb)
    _prime_scatters(acc, rows, didx1, semS)

    def run(r0, nrows, sidx, didx):
        _block_pipe(ei2d_ref, acc, cN, r0, nrows, x2_ref, rows, sidx, didx,
                    semi, semG, semS)

    def pairq(q, _):
        run(s * 80 + (2 * q) * 8, 8, sidx0, didx0)
        run(s * 80 + (2 * q + 1) * 8, 8, sidx1, didx1)
        return 0

    lax.fori_loop(0, 3, pairq, 0)

    @pl.when(s < 15)
    def _():
        def pairq2(q, _):
            run(s * 80 + (2 * q) * 8, 8, sidx0, didx0)
            run(s * 80 + (2 * q + 1) * 8, 8, sidx1, didx1)
            return 0

        lax.fori_loop(3, 5, pairq2, 0)

    @pl.when(s == 15)
    def _():
        run(ER - 2, 2, sidx1, didx1)

    _w_scat(x2_ref, rows, semS, 0)
    _w_scat(x2_ref, rows, semS, 1)
    plsc.subcore_barrier()
    pltpu.sync_copy(acc.at[pl.ds(base_row, ROWS_PER_TILE)],
                    out_ref.at[pl.ds(cN + base_row, ROWS_PER_TILE)])


_sc_spmm1 = functools.partial(
    pl.kernel,
    _spmm1_body,
    out_type=jax.ShapeDtypeStruct((2 * NP, 128), jnp.float32),
    mesh=_MESH,
    scratch_types=_spmm_scratch(),
)


def _spmm2_body(z_ref, ei2d_ref, out_ref, rows_a, rows_b, sidx0, didx0,
                sidx1, didx1, acc, semi, sga, sgb, ssa, ssb):
    c = lax.axis_index("c")
    s = lax.axis_index("s")
    base_row = s * ROWS_PER_TILE
    _spmm_init(rows_a, rows_b, didx1, acc, base_row)
    plsc.subcore_barrier()

    w = c * NS + s
    rows = (rows_a, rows_b)
    semG = (sga, sgb)
    semS = (ssa, ssb)
    _prime_scatters(acc, rows, didx1, semS)

    def run(r0, nrows, sidx, didx):
        _block_pipe(ei2d_ref, acc, None, r0, nrows, z_ref, rows, sidx, didx,
                    semi, semG, semS)

    run(w * 40, 8, sidx0, didx0)

    @pl.when(w < 31)
    def _():
        def pairq(q, _):
            run(w * 40 + (2 * q + 1) * 8, 8, sidx1, didx1)
            run(w * 40 + (2 * q + 2) * 8, 8, sidx0, didx0)
            return 0

        lax.fori_loop(0, 2, pairq, 0)

    @pl.when(w == 31)
    def _():
        run(ER - 2, 2, sidx1, didx1)

    _w_scat(z_ref, rows, semS, 0)
    _w_scat(z_ref, rows, semS, 1)
    plsc.subcore_barrier()
    pltpu.sync_copy(acc.at[pl.ds(base_row, ROWS_PER_TILE)],
                    out_ref.at[pl.ds(c * NP + base_row, ROWS_PER_TILE)])


_sc_spmm2 = functools.partial(
    pl.kernel,
    _spmm2_body,
    out_type=jax.ShapeDtypeStruct((2 * NP, 128), jnp.float32),
    mesh=_MESH,
    scratch_types=_spmm_scratch(),
)


BN = 2000


def _norms(deg_blk):
    ns = lax.rsqrt(jnp.maximum(deg_blk[0, :, 0:1], 1.0))
    nd = lax.rsqrt(jnp.maximum(deg_blk[1, :, 0:1], 1.0))
    return ns, nd


def _tc_prep_body(deg_ref, x_ref, o_ref):
    ns, _ = _norms(deg_ref[...])
    xn = x_ref[...] * ns
    o_ref[0] = xn[:, :128]
    o_ref[1] = xn[:, 128:]


def _tc_prep(deg2, x):
    return pl.pallas_call(
        _tc_prep_body,
        grid=(N // BN,),
        in_specs=[
            pl.BlockSpec((2, BN, 16), lambda i: (0, i, 0)),
            pl.BlockSpec((BN, D_IN), lambda i: (i, 0)),
        ],
        out_specs=pl.BlockSpec((2, BN, 128), lambda i: (0, i, 0)),
        out_shape=jax.ShapeDtypeStruct((2, NP, 128), jnp.float32),
    )(deg2, x)


def _tc_mid_body(deg_ref, a_ref, w1_ref, b1_ref, w2_ref, z_ref):
    ns, nd = _norms(deg_ref[...])
    a0 = a_ref[0] * nd
    a1 = a_ref[1] * nd
    h = jnp.dot(a0, w1_ref[0:128, :], preferred_element_type=jnp.float32)
    h += jnp.dot(a1, w1_ref[128:256, :], preferred_element_type=jnp.float32)
    h = jnp.maximum(h + b1_ref[...], 0.0)
    z_ref[...] = jnp.dot(h * ns, w2_ref[...], preferred_element_type=jnp.float32)


def _tc_mid(deg2, agg1, W1, b1, W2):
    return pl.pallas_call(
        _tc_mid_body,
        grid=(N // BN,),
        in_specs=[
            pl.BlockSpec((2, BN, 16), lambda i: (0, i, 0)),
            pl.BlockSpec((2, BN, 128), lambda i: (0, i, 0)),
            pl.BlockSpec((D_IN, D_H), lambda i: (0, 0)),
            pl.BlockSpec((1, D_H), lambda i: (0, 0)),
            pl.BlockSpec((D_H, 128), lambda i: (0, 0)),
        ],
        out_specs=pl.BlockSpec((BN, 128), lambda i: (i, 0)),
        out_shape=jax.ShapeDtypeStruct((NP, 128), jnp.float32),
    )(deg2, agg1, W1, b1, W2)


def _tc_fin_body(deg_ref, az_ref, b2_ref, h_ref, p_ref):
    _, nd = _norms(deg_ref[...])
    h = (az_ref[0] + az_ref[1])[:, :D_OUT] * nd + b2_ref[...]
    h_ref[...] = h

    @pl.when(pl.program_id(0) == 0)
    def _():
        p_ref[...] = jnp.zeros_like(p_ref)

    p_ref[...] += jnp.sum(h, axis=0, keepdims=True) * (1.0 / N)


def _tc_fin(deg2, aggz, b2):
    return pl.pallas_call(
        _tc_fin_body,
        grid=(N // BN,),
        in_specs=[
            pl.BlockSpec((2, BN, 16), lambda i: (0, i, 0)),
            pl.BlockSpec((2, BN, 128), lambda i: (0, i, 0)),
            pl.BlockSpec((1, D_OUT), lambda i: (0, 0)),
        ],
        out_specs=[
            pl.BlockSpec((BN, D_OUT), lambda i: (i, 0)),
            pl.BlockSpec((1, D_OUT), lambda i: (0, 0)),
        ],
        out_shape=[
            jax.ShapeDtypeStruct((N, D_OUT), jnp.float32),
            jax.ShapeDtypeStruct((1, D_OUT), jnp.float32),
        ],
    )(deg2, aggz, b2)


def kernel(in_feat, edge_index, W1, b1, W2, b2):
    ei32 = edge_index.astype(jnp.int32)
    pad = jnp.zeros(((ERP - ER) * CHUNK,), jnp.int32)
    eiflat = jnp.concatenate([ei32[0], pad, ei32[1], pad])
    ei2d = eiflat.reshape(2 * ERP, CHUNK)

    deg2 = _sc_degree()(eiflat).reshape(2, NP, 16)
    xs = _tc_prep(deg2, in_feat)
    agg1 = _sc_spmm1()(xs.reshape(2 * NP, 128), ei2d)
    W2p = jnp.pad(W2, ((0, 0), (0, 128 - D_OUT)))
    z = _tc_mid(deg2, agg1.reshape(2, NP, 128), W1, b1.reshape(1, D_H), W2p)
    aggz = _sc_spmm2()(z, ei2d)
    h2, pooled = _tc_fin(deg2, aggz.reshape(2, NP, 128), b2.reshape(1, D_OUT))
    return (h2, pooled)
